# baseline (speedup 1.0000x reference)
import functools

import jax
import jax.numpy as jnp
from jax import lax
from jax.experimental import pallas as pl
from jax.experimental.pallas import tpu as pltpu
from jax.experimental.pallas import tpu_sc as plsc

N = 10000
E = 320000
D_IN = 128
D_H = 64

NC = 2
NS = 16
NW = NC * NS
K = 64
NJ = 158
EPW = NJ * K
E_PAD = NW * EPW
NP = 10016
RPT = 640
L = 16
ZC = 40
NBUF = 2

_mesh = plsc.VectorSubcoreMesh(core_axis_name="c", subcore_axis_name="s")
_sc_params = pltpu.CompilerParams(use_tc_tiling_on_sc=False)


@functools.partial(
    pl.kernel,
    out_type=jax.ShapeDtypeStruct((NC, N, L), jnp.float32),
    mesh=_mesh,
    compiler_params=_sc_params,
    scratch_types=[
        pltpu.VMEM((NJ, K), jnp.int32),
        pltpu.VMEM((K, L), jnp.float32),
        pltpu.VMEM_SHARED((NP, L), jnp.float32),
    ],
)
def _deg_kernel(dst_hbm, out_hbm, dst_v, ones_v, acc):
    c = lax.axis_index("c")
    s = lax.axis_index("s")
    w = s * NC + c
    pltpu.sync_copy(dst_hbm.at[w], dst_v)

    def fill(i, carry):
        ones_v[i, :] = jnp.full((L,), carry, jnp.float32)
        return carry

    lax.fori_loop(0, K, fill, 0.0)
    base = s * RPT
    for m in range(RPT // ZC):
        if (m + 1) * ZC <= 400:
            pltpu.sync_copy(ones_v.at[pl.ds(0, ZC)],
                            acc.at[pl.ds(base + m * ZC, ZC)])
        else:
            @pl.when(s < NS - 1)
            def _():
                pltpu.sync_copy(ones_v.at[pl.ds(0, ZC)],
                                acc.at[pl.ds(base + m * ZC, ZC)])
    lax.fori_loop(0, K, fill, 1.0)
    plsc.subcore_barrier()

    def body(j, carry):
        pltpu.sync_copy(ones_v, acc.at[dst_v.at[j]], add=True)
        return carry

    lax.fori_loop(0, NJ, body, 0)
    plsc.subcore_barrier()

    @pl.when(s < NS - 1)
    def _():
        pltpu.sync_copy(acc.at[pl.ds(base, RPT)], out_hbm.at[c, pl.ds(base, RPT)])

    @pl.when(s == NS - 1)
    def _():
        pltpu.sync_copy(acc.at[pl.ds(N - 400, 400)],
                        out_hbm.at[c, pl.ds(N - 400, 400)])


@functools.partial(
    pl.kernel,
    out_type=jax.ShapeDtypeStruct((NC, N, D_H), jnp.float32),
    mesh=_mesh,
    compiler_params=_sc_params,
    scratch_types=[
        pltpu.VMEM((NJ, K), jnp.int32),
        pltpu.VMEM((NJ, K), jnp.int32),
        [pltpu.VMEM((K, D_H), jnp.float32)] * NBUF,
        pltpu.VMEM_SHARED((NP, D_H), jnp.float32),
        [pltpu.SemaphoreType.DMA] * NBUF,
    ],
)
def _edge_kernel(hs_hbm, src_hbm, dst_hbm, out_hbm, src_v, dst_v, rows,
                 acc, semg):
    c = lax.axis_index("c")
    s = lax.axis_index("s")
    w = s * NC + c

    pltpu.sync_copy(src_hbm.at[w], src_v)
    pltpu.sync_copy(dst_hbm.at[w], dst_v)

    zero = jnp.zeros((L,), jnp.float32)

    def zbody(i, carry):
        def zcol(k2, carry2):
            rows[0][i, pl.ds(k2 * L, L)] = zero
            return carry2

        return lax.fori_loop(0, D_H // L, zcol, carry)

    lax.fori_loop(0, K, zbody, 0)

    base = s * RPT
    for m in range(RPT // ZC):
        if (m + 1) * ZC <= 400:
            pltpu.sync_copy(rows[0].at[pl.ds(0, ZC)],
                            acc.at[pl.ds(base + m * ZC, ZC)])
        else:
            @pl.when(s < NS - 1)
            def _():
                pltpu.sync_copy(rows[0].at[pl.ds(0, ZC)],
                                acc.at[pl.ds(base + m * ZC, ZC)])
    plsc.subcore_barrier()

    def gwait(j, b):
        pltpu.make_async_copy(hs_hbm.at[src_v.at[j]], rows[b], semg[b]).wait()

    pltpu.async_copy(hs_hbm.at[src_v.at[0]], rows[0], semg[0])

    def body(i, carry):
        j = 2 * i
        gwait(j, 0)
        pltpu.async_copy(hs_hbm.at[src_v.at[j + 1]], rows[1], semg[1])
        pltpu.sync_copy(rows[0], acc.at[dst_v.at[j]], add=True)
        gwait(j + 1, 1)
        pltpu.async_copy(hs_hbm.at[src_v.at[j + 2]], rows[0], semg[0])
        pltpu.sync_copy(rows[1], acc.at[dst_v.at[j + 1]], add=True)
        return carry

    lax.fori_loop(0, NJ // 2 - 1, body, 0)
    gwait(NJ - 2, 0)
    pltpu.async_copy(hs_hbm.at[src_v.at[NJ - 1]], rows[1], semg[1])
    pltpu.sync_copy(rows[0], acc.at[dst_v.at[NJ - 2]], add=True)
    gwait(NJ - 1, 1)
    pltpu.sync_copy(rows[1], acc.at[dst_v.at[NJ - 1]], add=True)
    plsc.subcore_barrier()

    @pl.when(s < NS - 1)
    def _():
        pltpu.sync_copy(acc.at[pl.ds(base, RPT)], out_hbm.at[c, pl.ds(base, RPT)])

    @pl.when(s == NS - 1)
    def _():
        pltpu.sync_copy(acc.at[pl.ds(N - 400, 400)],
                        out_hbm.at[c, pl.ds(N - 400, 400)])


def _tc_first_body(hist_ref, x_ref, w0_ref, dinv_ref, hs_ref):
    deg = hist_ref[0, :, 0:1] + hist_ref[1, :, 0:1] + 1.0
    dinv = lax.rsqrt(deg)
    h = jnp.dot(x_ref[...], w0_ref[...], preferred_element_type=jnp.float32)
    dinv_ref[...] = dinv
    hs_ref[...] = dinv * h


def _tc_mid_body(acc_ref, hs_ref, dinv_ref, b_ref, gw_ref, gb_ref, ga_ref,
                 wn_ref, hsn_ref):
    dinv = dinv_ref[...]
    sacc = acc_ref[0] + acc_ref[1] + hs_ref[...]
    conv = dinv * sacc + b_ref[...]
    mean = jnp.mean(conv, axis=0, keepdims=True)
    xc = conv - ga_ref[...] * mean
    var = jnp.mean(xc * xc, axis=0, keepdims=True)
    g = gw_ref[...] * xc * lax.rsqrt(var + 1e-5) + gb_ref[...]
    hsn_ref[...] = dinv * jnp.dot(g, wn_ref[...],
                                  preferred_element_type=jnp.float32)


def _tc_last_body(acc_ref, hs_ref, dinv_ref, b_ref, gw_ref, gb_ref, ga_ref,
                  out_ref):
    sacc = acc_ref[0] + acc_ref[1] + hs_ref[...]
    conv = dinv_ref[...] * sacc + b_ref[...]
    mean = jnp.mean(conv, axis=0, keepdims=True)
    xc = conv - ga_ref[...] * mean
    var = jnp.mean(xc * xc, axis=0, keepdims=True)
    out_ref[...] = gw_ref[...] * xc * lax.rsqrt(var + 1e-5) + gb_ref[...]


_f32 = jnp.float32
_tc_first = pl.pallas_call(
    _tc_first_body,
    out_shape=[jax.ShapeDtypeStruct((N, 1), _f32),
               jax.ShapeDtypeStruct((N, D_H), _f32)],
)
_tc_mid = pl.pallas_call(
    _tc_mid_body,
    out_shape=jax.ShapeDtypeStruct((N, D_H), _f32),
)
_tc_last = pl.pallas_call(
    _tc_last_body,
    out_shape=jax.ShapeDtypeStruct((N, D_H), _f32),
)


def kernel(x, edge_index, W0, b0, gw0, gb0, ga0, W1, b1, gw1, gb1, ga1,
           W2, b2, gw2, gb2, ga2):
    pad = E_PAD - E
    src_r = jnp.concatenate(
        [edge_index[0], jnp.zeros((pad,), jnp.int32)]).reshape(NW, NJ, K)
    dst_r = jnp.concatenate(
        [edge_index[1], jnp.full((pad,), N, jnp.int32)]).reshape(NW, NJ, K)

    histp = _deg_kernel(dst_r)
    dinv, hs = _tc_first(histp, x, W0)

    params = [(b0, gw0, gb0, ga0), (b1, gw1, gb1, ga1), (b2, gw2, gb2, ga2)]
    row = lambda v: v.reshape(1, D_H)

    for layer in range(3):
        acc = _edge_kernel(hs, src_r, dst_r)
        b, gw, gb, ga = (row(v) for v in params[layer])
        if layer < 2:
            wn = (W1, W2)[layer]
            hs = _tc_mid(acc, hs, dinv, b, gw, gb, ga, wn)
        else:
            out = _tc_last(acc, hs, dinv, b, gw, gb, ga)
    return out


# speedup vs baseline: 1.1866x; 1.1866x over previous
import functools

import jax
import jax.numpy as jnp
from jax import lax
from jax.experimental import pallas as pl
from jax.experimental.pallas import tpu as pltpu
from jax.experimental.pallas import tpu_sc as plsc

N = 10000
E = 320000
D_IN = 128
D_H = 64

NC = 2
NS = 16
NW = NC * NS
K = 80
NJ = 126
EPW = NJ * K
E_PAD = NW * EPW
NP = 10016
RPT = 640
L = 16
ZC = 80
NBUF = 2

_mesh = plsc.VectorSubcoreMesh(core_axis_name="c", subcore_axis_name="s")
_sc_params = pltpu.CompilerParams(use_tc_tiling_on_sc=False)


@functools.partial(
    pl.kernel,
    out_type=jax.ShapeDtypeStruct((NC, N, L), jnp.float32),
    mesh=_mesh,
    compiler_params=_sc_params,
    scratch_types=[
        pltpu.VMEM((NJ, K), jnp.int32),
        pltpu.VMEM((K, L), jnp.float32),
        pltpu.VMEM_SHARED((NP, L), jnp.float32),
    ],
)
def _deg_kernel(dst_hbm, out_hbm, dst_v, ones_v, acc):
    c = lax.axis_index("c")
    s = lax.axis_index("s")
    w = s * NC + c
    pltpu.sync_copy(dst_hbm.at[w], dst_v)

    def fill(i, carry):
        ones_v[i, :] = jnp.full((L,), carry, jnp.float32)
        return carry

    lax.fori_loop(0, K, fill, 0.0)
    base = s * RPT
    for m in range(RPT // ZC):
        if (m + 1) * ZC <= 400:
            pltpu.sync_copy(ones_v.at[pl.ds(0, ZC)],
                            acc.at[pl.ds(base + m * ZC, ZC)])
        else:
            @pl.when(s < NS - 1)
            def _():
                pltpu.sync_copy(ones_v.at[pl.ds(0, ZC)],
                                acc.at[pl.ds(base + m * ZC, ZC)])
    lax.fori_loop(0, K, fill, 1.0)
    plsc.subcore_barrier()

    def body(j, carry):
        pltpu.sync_copy(ones_v, acc.at[dst_v.at[j]], add=True)
        return carry

    lax.fori_loop(0, NJ, body, 0)
    plsc.subcore_barrier()

    @pl.when(s < NS - 1)
    def _():
        pltpu.sync_copy(acc.at[pl.ds(base, RPT)], out_hbm.at[c, pl.ds(base, RPT)])

    @pl.when(s == NS - 1)
    def _():
        pltpu.sync_copy(acc.at[pl.ds(N - 400, 400)],
                        out_hbm.at[c, pl.ds(N - 400, 400)])


@functools.partial(
    pl.kernel,
    out_type=jax.ShapeDtypeStruct((NC, N, D_H), jnp.float32),
    mesh=_mesh,
    compiler_params=_sc_params,
    scratch_types=[
        pltpu.VMEM((NJ, K), jnp.int32),
        pltpu.VMEM((NJ, K), jnp.int32),
        [pltpu.VMEM((K, D_H), jnp.float32)] * NBUF,
        pltpu.VMEM_SHARED((NP, D_H), jnp.float32),
        [pltpu.SemaphoreType.DMA] * NBUF,
    ],
)
def _edge_kernel(hs_hbm, src_hbm, dst_hbm, out_hbm, src_v, dst_v, rows,
                 acc, semg):
    c = lax.axis_index("c")
    s = lax.axis_index("s")
    w = s * NC + c

    pltpu.sync_copy(src_hbm.at[w], src_v)
    pltpu.sync_copy(dst_hbm.at[w], dst_v)

    zero = jnp.zeros((L,), jnp.float32)

    def zbody(i, carry):
        def zcol(k2, carry2):
            rows[0][i, pl.ds(k2 * L, L)] = zero
            return carry2

        return lax.fori_loop(0, D_H // L, zcol, carry)

    lax.fori_loop(0, K, zbody, 0)

    base = s * RPT
    for m in range(RPT // ZC):
        if (m + 1) * ZC <= 400:
            pltpu.sync_copy(rows[0].at[pl.ds(0, ZC)],
                            acc.at[pl.ds(base + m * ZC, ZC)])
        else:
            @pl.when(s < NS - 1)
            def _():
                pltpu.sync_copy(rows[0].at[pl.ds(0, ZC)],
                                acc.at[pl.ds(base + m * ZC, ZC)])
    plsc.subcore_barrier()

    def gwait(j, b):
        pltpu.make_async_copy(hs_hbm.at[src_v.at[j]], rows[b], semg[b]).wait()

    pltpu.async_copy(hs_hbm.at[src_v.at[0]], rows[0], semg[0])

    def body(i, carry):
        j = 2 * i
        gwait(j, 0)
        pltpu.async_copy(hs_hbm.at[src_v.at[j + 1]], rows[1], semg[1])
        pltpu.sync_copy(rows[0], acc.at[dst_v.at[j]], add=True)
        gwait(j + 1, 1)
        pltpu.async_copy(hs_hbm.at[src_v.at[j + 2]], rows[0], semg[0])
        pltpu.sync_copy(rows[1], acc.at[dst_v.at[j + 1]], add=True)
        return carry

    lax.fori_loop(0, NJ // 2 - 1, body, 0)
    gwait(NJ - 2, 0)
    pltpu.async_copy(hs_hbm.at[src_v.at[NJ - 1]], rows[1], semg[1])
    pltpu.sync_copy(rows[0], acc.at[dst_v.at[NJ - 2]], add=True)
    gwait(NJ - 1, 1)
    pltpu.sync_copy(rows[1], acc.at[dst_v.at[NJ - 1]], add=True)
    plsc.subcore_barrier()

    @pl.when(s < NS - 1)
    def _():
        pltpu.sync_copy(acc.at[pl.ds(base, RPT)], out_hbm.at[c, pl.ds(base, RPT)])

    @pl.when(s == NS - 1)
    def _():
        pltpu.sync_copy(acc.at[pl.ds(N - 400, 400)],
                        out_hbm.at[c, pl.ds(N - 400, 400)])


def _tc_first_body(hist_ref, x_ref, w0_ref, dinv_ref, hs_ref):
    deg = hist_ref[0, :, 0:1] + hist_ref[1, :, 0:1] + 1.0
    dinv = lax.rsqrt(deg)
    h = jnp.dot(x_ref[...], w0_ref[...], preferred_element_type=jnp.float32)
    dinv_ref[...] = dinv
    hs_ref[...] = dinv * h


def _tc_mid_body(acc_ref, hs_ref, dinv_ref, b_ref, gw_ref, gb_ref, ga_ref,
                 wn_ref, hsn_ref):
    dinv = dinv_ref[...]
    sacc = acc_ref[0] + acc_ref[1] + hs_ref[...]
    conv = dinv * sacc + b_ref[...]
    mean = jnp.mean(conv, axis=0, keepdims=True)
    xc = conv - ga_ref[...] * mean
    var = jnp.mean(xc * xc, axis=0, keepdims=True)
    g = gw_ref[...] * xc * lax.rsqrt(var + 1e-5) + gb_ref[...]
    hsn_ref[...] = dinv * jnp.dot(g, wn_ref[...],
                                  preferred_element_type=jnp.float32)


def _tc_last_body(acc_ref, hs_ref, dinv_ref, b_ref, gw_ref, gb_ref, ga_ref,
                  out_ref):
    sacc = acc_ref[0] + acc_ref[1] + hs_ref[...]
    conv = dinv_ref[...] * sacc + b_ref[...]
    mean = jnp.mean(conv, axis=0, keepdims=True)
    xc = conv - ga_ref[...] * mean
    var = jnp.mean(xc * xc, axis=0, keepdims=True)
    out_ref[...] = gw_ref[...] * xc * lax.rsqrt(var + 1e-5) + gb_ref[...]


_f32 = jnp.float32
_tc_first = pl.pallas_call(
    _tc_first_body,
    out_shape=[jax.ShapeDtypeStruct((N, 1), _f32),
               jax.ShapeDtypeStruct((N, D_H), _f32)],
)
_tc_mid = pl.pallas_call(
    _tc_mid_body,
    out_shape=jax.ShapeDtypeStruct((N, D_H), _f32),
)
_tc_last = pl.pallas_call(
    _tc_last_body,
    out_shape=jax.ShapeDtypeStruct((N, D_H), _f32),
)


def kernel(x, edge_index, W0, b0, gw0, gb0, ga0, W1, b1, gw1, gb1, ga1,
           W2, b2, gw2, gb2, ga2):
    pad = E_PAD - E
    src_r = jnp.concatenate(
        [edge_index[0], jnp.zeros((pad,), jnp.int32)]).reshape(NW, NJ, K)
    dst_r = jnp.concatenate(
        [edge_index[1], jnp.full((pad,), N, jnp.int32)]).reshape(NW, NJ, K)

    histp = _deg_kernel(dst_r)
    dinv, hs = _tc_first(histp, x, W0)

    params = [(b0, gw0, gb0, ga0), (b1, gw1, gb1, ga1), (b2, gw2, gb2, ga2)]
    row = lambda v: v.reshape(1, D_H)

    for layer in range(3):
        acc = _edge_kernel(hs, src_r, dst_r)
        b, gw, gb, ga = (row(v) for v in params[layer])
        if layer < 2:
            wn = (W1, W2)[layer]
            hs = _tc_mid(acc, hs, dinv, b, gw, gb, ga, wn)
        else:
            out = _tc_last(acc, hs, dinv, b, gw, gb, ga)
    return out


# speedup vs baseline: 1.5266x; 1.2865x over previous
import functools

import jax
import jax.numpy as jnp
from jax import lax
from jax.experimental import pallas as pl
from jax.experimental.pallas import tpu as pltpu
from jax.experimental.pallas import tpu_sc as plsc

N = 10000
E = 320000
D_IN = 128
D_H = 64

NC = 2
NS = 16
NW = NC * NS
K = 80
NJ = 126
EPW = NJ * K
E_PAD = NW * EPW
NP = 10240
RPT = 640
L = 16
ZC = 80
NBUF = 2

_mesh = plsc.VectorSubcoreMesh(core_axis_name="c", subcore_axis_name="s")
_sc_params = pltpu.CompilerParams(use_tc_tiling_on_sc=False)


@functools.partial(
    pl.kernel,
    out_type=jax.ShapeDtypeStruct((NC, N, L), jnp.float32),
    mesh=_mesh,
    compiler_params=_sc_params,
    scratch_types=[
        pltpu.VMEM((NJ, K), jnp.int32),
        pltpu.VMEM((K, L), jnp.float32),
        pltpu.VMEM_SHARED((NP, L), jnp.float32),
    ],
)
def _deg_kernel(dst_hbm, out_hbm, dst_v, ones_v, acc):
    c = lax.axis_index("c")
    s = lax.axis_index("s")
    w = s * NC + c
    pltpu.sync_copy(dst_hbm.at[w], dst_v)

    def fill(i, carry):
        ones_v[i, :] = jnp.full((L,), carry, jnp.float32)
        return carry

    lax.fori_loop(0, K, fill, 0.0)
    base = s * RPT
    for m in range(RPT // ZC):
        if (m + 1) * ZC <= 400:
            pltpu.sync_copy(ones_v.at[pl.ds(0, ZC)],
                            acc.at[pl.ds(base + m * ZC, ZC)])
        else:
            @pl.when(s < NS - 1)
            def _():
                pltpu.sync_copy(ones_v.at[pl.ds(0, ZC)],
                                acc.at[pl.ds(base + m * ZC, ZC)])
    lax.fori_loop(0, K, fill, 1.0)
    plsc.subcore_barrier()

    def body(j, carry):
        pltpu.sync_copy(ones_v, acc.at[dst_v.at[j]], add=True)
        return carry

    lax.fori_loop(0, NJ, body, 0)
    plsc.subcore_barrier()

    @pl.when(s < NS - 1)
    def _():
        pltpu.sync_copy(acc.at[pl.ds(base, RPT)], out_hbm.at[c, pl.ds(base, RPT)])

    @pl.when(s == NS - 1)
    def _():
        pltpu.sync_copy(acc.at[pl.ds(N - 400, 400)],
                        out_hbm.at[c, pl.ds(N - 400, 400)])


@functools.partial(
    pl.kernel,
    out_type=jax.ShapeDtypeStruct((NC, N, D_H), jnp.float32),
    mesh=_mesh,
    compiler_params=_sc_params,
    scratch_types=[
        pltpu.VMEM((NJ, K), jnp.int32),
        pltpu.VMEM((NJ, K), jnp.int32),
        [pltpu.VMEM((K, D_H), jnp.float32)] * NBUF,
        pltpu.VMEM_SHARED((NP, D_H), jnp.float32),
        [pltpu.SemaphoreType.DMA] * NBUF,
    ],
)
def _edge_kernel(hs_hbm, src_hbm, dst_hbm, out_hbm, src_v, dst_v, rows,
                 acc, semg):
    c = lax.axis_index("c")
    s = lax.axis_index("s")
    w = s * NC + c

    pltpu.sync_copy(src_hbm.at[w], src_v)
    pltpu.sync_copy(dst_hbm.at[w], dst_v)

    zero = jnp.zeros((L,), jnp.float32)

    def zbody(i, carry):
        def zcol(k2, carry2):
            rows[0][i, pl.ds(k2 * L, L)] = zero
            return carry2

        return lax.fori_loop(0, D_H // L, zcol, carry)

    lax.fori_loop(0, K, zbody, 0)

    base = s * RPT
    for m in range(RPT // ZC):
        if (m + 1) * ZC <= 400:
            pltpu.sync_copy(rows[0].at[pl.ds(0, ZC)],
                            acc.at[pl.ds(base + m * ZC, ZC)])
        else:
            @pl.when(s < NS - 1)
            def _():
                pltpu.sync_copy(rows[0].at[pl.ds(0, ZC)],
                                acc.at[pl.ds(base + m * ZC, ZC)])
    plsc.subcore_barrier()

    def gwait(j, b):
        pltpu.make_async_copy(hs_hbm.at[src_v.at[j]], rows[b], semg[b]).wait()

    pltpu.async_copy(hs_hbm.at[src_v.at[0]], rows[0], semg[0])

    def body(i, carry):
        j = 2 * i
        gwait(j, 0)
        pltpu.async_copy(hs_hbm.at[src_v.at[j + 1]], rows[1], semg[1])
        pltpu.sync_copy(rows[0], acc.at[dst_v.at[j]], add=True)
        gwait(j + 1, 1)
        pltpu.async_copy(hs_hbm.at[src_v.at[j + 2]], rows[0], semg[0])
        pltpu.sync_copy(rows[1], acc.at[dst_v.at[j + 1]], add=True)
        return carry

    lax.fori_loop(0, NJ // 2 - 1, body, 0)
    gwait(NJ - 2, 0)
    pltpu.async_copy(hs_hbm.at[src_v.at[NJ - 1]], rows[1], semg[1])
    pltpu.sync_copy(rows[0], acc.at[dst_v.at[NJ - 2]], add=True)
    gwait(NJ - 1, 1)
    pltpu.sync_copy(rows[1], acc.at[dst_v.at[NJ - 1]], add=True)
    plsc.subcore_barrier()

    @pl.when(s < NS - 1)
    def _():
        pltpu.sync_copy(acc.at[pl.ds(base, RPT)], out_hbm.at[c, pl.ds(base, RPT)])

    @pl.when(s == NS - 1)
    def _():
        pltpu.sync_copy(acc.at[pl.ds(N - 400, 400)],
                        out_hbm.at[c, pl.ds(N - 400, 400)])


def _tc_first_body(hist_ref, x_ref, w0_ref, dinv_ref, hs_ref):
    deg = hist_ref[0, :, 0:1] + hist_ref[1, :, 0:1] + 1.0
    dinv = lax.rsqrt(deg)
    h = jnp.dot(x_ref[...], w0_ref[...], preferred_element_type=jnp.float32)
    dinv_ref[...] = dinv
    hs_ref[...] = dinv * h


def _tc_mid_body(acc_ref, hs_ref, dinv_ref, b_ref, gw_ref, gb_ref, ga_ref,
                 wn_ref, hsn_ref):
    dinv = dinv_ref[...]
    sacc = acc_ref[0] + acc_ref[1] + hs_ref[...]
    conv = dinv * sacc + b_ref[...]
    mean = jnp.mean(conv, axis=0, keepdims=True)
    xc = conv - ga_ref[...] * mean
    var = jnp.mean(xc * xc, axis=0, keepdims=True)
    g = gw_ref[...] * xc * lax.rsqrt(var + 1e-5) + gb_ref[...]
    hsn_ref[...] = dinv * jnp.dot(g, wn_ref[...],
                                  preferred_element_type=jnp.float32)


def _tc_last_body(acc_ref, hs_ref, dinv_ref, b_ref, gw_ref, gb_ref, ga_ref,
                  out_ref):
    sacc = acc_ref[0] + acc_ref[1] + hs_ref[...]
    conv = dinv_ref[...] * sacc + b_ref[...]
    mean = jnp.mean(conv, axis=0, keepdims=True)
    xc = conv - ga_ref[...] * mean
    var = jnp.mean(xc * xc, axis=0, keepdims=True)
    out_ref[...] = gw_ref[...] * xc * lax.rsqrt(var + 1e-5) + gb_ref[...]


_f32 = jnp.float32
_tc_first = pl.pallas_call(
    _tc_first_body,
    out_shape=[jax.ShapeDtypeStruct((N, 1), _f32),
               jax.ShapeDtypeStruct((N, D_H), _f32)],
)
_tc_mid = pl.pallas_call(
    _tc_mid_body,
    out_shape=jax.ShapeDtypeStruct((N, D_H), _f32),
)
_tc_last = pl.pallas_call(
    _tc_last_body,
    out_shape=jax.ShapeDtypeStruct((N, D_H), _f32),
)


def kernel(x, edge_index, W0, b0, gw0, gb0, ga0, W1, b1, gw1, gb1, ga1,
           W2, b2, gw2, gb2, ga2):
    pad = E_PAD - E
    pidx = jnp.arange(pad, dtype=jnp.int32)
    src_r = jnp.concatenate(
        [edge_index[0], pidx % N]).reshape(NW, NJ, K)
    dst_r = jnp.concatenate(
        [edge_index[1], N + pidx % (NP - N)]).reshape(NW, NJ, K)

    histp = _deg_kernel(dst_r)
    dinv, hs = _tc_first(histp, x, W0)

    params = [(b0, gw0, gb0, ga0), (b1, gw1, gb1, ga1), (b2, gw2, gb2, ga2)]
    row = lambda v: v.reshape(1, D_H)

    for layer in range(3):
        acc = _edge_kernel(hs, src_r, dst_r)
        b, gw, gb, ga = (row(v) for v in params[layer])
        if layer < 2:
            wn = (W1, W2)[layer]
            hs = _tc_mid(acc, hs, dinv, b, gw, gb, ga, wn)
        else:
            out = _tc_last(acc, hs, dinv, b, gw, gb, ga)
    return out


# speedup vs baseline: 2.0593x; 1.3490x over previous
import functools

import jax
import jax.numpy as jnp
from jax import lax
from jax.experimental import pallas as pl
from jax.experimental.pallas import tpu as pltpu
from jax.experimental.pallas import tpu_sc as plsc

N = 10000
E = 320000
D_IN = 128
D_H = 64

NC = 2
NS = 16
NW = NC * NS
K = 80
NJ = 126
EPW = NJ * K
E_PAD = NW * EPW
NP = 10240
RPT = 640
L = 16
ZC = 80
NBUF = 4

_mesh = plsc.VectorSubcoreMesh(core_axis_name="c", subcore_axis_name="s")
_sc_params = pltpu.CompilerParams(use_tc_tiling_on_sc=False)


@functools.partial(
    pl.kernel,
    out_type=jax.ShapeDtypeStruct((NC, N, L), jnp.float32),
    mesh=_mesh,
    compiler_params=_sc_params,
    scratch_types=[
        pltpu.VMEM((NJ, K), jnp.int32),
        pltpu.VMEM((K, L), jnp.float32),
        pltpu.VMEM_SHARED((NP, L), jnp.float32),
    ],
)
def _deg_kernel(dst_hbm, out_hbm, dst_v, ones_v, acc):
    c = lax.axis_index("c")
    s = lax.axis_index("s")
    w = s * NC + c
    pltpu.sync_copy(dst_hbm.at[w], dst_v)

    def fill(i, carry):
        ones_v[i, :] = jnp.full((L,), carry, jnp.float32)
        return carry

    lax.fori_loop(0, K, fill, 0.0)
    base = s * RPT
    for m in range(RPT // ZC):
        if (m + 1) * ZC <= 400:
            pltpu.sync_copy(ones_v.at[pl.ds(0, ZC)],
                            acc.at[pl.ds(base + m * ZC, ZC)])
        else:
            @pl.when(s < NS - 1)
            def _():
                pltpu.sync_copy(ones_v.at[pl.ds(0, ZC)],
                                acc.at[pl.ds(base + m * ZC, ZC)])
    lax.fori_loop(0, K, fill, 1.0)
    plsc.subcore_barrier()

    def body(j, carry):
        pltpu.sync_copy(ones_v, acc.at[dst_v.at[j]], add=True)
        return carry

    lax.fori_loop(0, NJ, body, 0)
    plsc.subcore_barrier()

    @pl.when(s < NS - 1)
    def _():
        pltpu.sync_copy(acc.at[pl.ds(base, RPT)], out_hbm.at[c, pl.ds(base, RPT)])

    @pl.when(s == NS - 1)
    def _():
        pltpu.sync_copy(acc.at[pl.ds(N - 400, 400)],
                        out_hbm.at[c, pl.ds(N - 400, 400)])


@functools.partial(
    pl.kernel,
    out_type=jax.ShapeDtypeStruct((NC, N, D_H), jnp.float32),
    mesh=_mesh,
    compiler_params=_sc_params,
    scratch_types=[
        pltpu.VMEM((NJ, K), jnp.int32),
        pltpu.VMEM((NJ, K), jnp.int32),
        [pltpu.VMEM((K, D_H), jnp.float32)] * NBUF,
        pltpu.VMEM_SHARED((NP, D_H), jnp.float32),
        [pltpu.SemaphoreType.DMA] * NBUF,
        [pltpu.SemaphoreType.DMA] * NBUF,
    ],
)
def _edge_kernel(hs_hbm, src_hbm, dst_hbm, out_hbm, src_v, dst_v, rows,
                 acc, semg, sems):
    c = lax.axis_index("c")
    s = lax.axis_index("s")
    w = s * NC + c

    pltpu.sync_copy(src_hbm.at[w], src_v)
    pltpu.sync_copy(dst_hbm.at[w], dst_v)

    zero = jnp.zeros((L,), jnp.float32)

    def zbody(i, carry):
        def zcol(k2, carry2):
            rows[0][i, pl.ds(k2 * L, L)] = zero
            return carry2

        return lax.fori_loop(0, D_H // L, zcol, carry)

    lax.fori_loop(0, K, zbody, 0)

    base = s * RPT
    for m in range(RPT // ZC):
        if (m + 1) * ZC <= 400:
            pltpu.sync_copy(rows[0].at[pl.ds(0, ZC)],
                            acc.at[pl.ds(base + m * ZC, ZC)])
        else:
            @pl.when(s < NS - 1)
            def _():
                pltpu.sync_copy(rows[0].at[pl.ds(0, ZC)],
                                acc.at[pl.ds(base + m * ZC, ZC)])
    plsc.subcore_barrier()

    def gwait(j, b):
        pltpu.make_async_copy(hs_hbm.at[src_v.at[j]], rows[b], semg[b]).wait()

    def swait(j, b):
        pltpu.make_async_copy(rows[b], acc.at[dst_v.at[j]], sems[b]).wait()

    def sfire(j, b):
        pltpu.async_copy(rows[b], acc.at[dst_v.at[j]], sems[b], add=True)

    def gfire(j, b):
        pltpu.async_copy(hs_hbm.at[src_v.at[j]], rows[b], semg[b])

    for b in range(2):
        gfire(b, b)
    for p in range(2):
        gwait(p, p)
        sfire(p, p)
        gfire(p + 2, p + 2)

    def body(i, carry):
        p0 = 2 + 4 * i
        for u in range(4):
            b = (2 + u) % 4
            bn = (b + 2) % 4
            gwait(p0 + u, b)
            sfire(p0 + u, b)
            swait(p0 + u - 2, bn)
            gfire(p0 + u + 2, bn)
        return carry

    lax.fori_loop(0, (NJ - 4) // 4, body, 0)
    for p in range(NJ - 4, NJ):
        b = p % 4
        gwait(p, b)
        sfire(p, b)
        if p + 2 < NJ:
            bn = (b + 2) % 4
            swait(p - 2, bn)
            gfire(p + 2, bn)
    for p in range(NJ - 4, NJ):
        swait(p, p % 4)
    plsc.subcore_barrier()

    @pl.when(s < NS - 1)
    def _():
        pltpu.sync_copy(acc.at[pl.ds(base, RPT)], out_hbm.at[c, pl.ds(base, RPT)])

    @pl.when(s == NS - 1)
    def _():
        pltpu.sync_copy(acc.at[pl.ds(N - 400, 400)],
                        out_hbm.at[c, pl.ds(N - 400, 400)])


def _tc_first_body(hist_ref, x_ref, w0_ref, dinv_ref, hs_ref):
    deg = hist_ref[0, :, 0:1] + hist_ref[1, :, 0:1] + 1.0
    dinv = lax.rsqrt(deg)
    h = jnp.dot(x_ref[...], w0_ref[...], preferred_element_type=jnp.float32)
    dinv_ref[...] = dinv
    hs_ref[...] = dinv * h


def _tc_mid_body(acc_ref, hs_ref, dinv_ref, b_ref, gw_ref, gb_ref, ga_ref,
                 wn_ref, hsn_ref):
    dinv = dinv_ref[...]
    sacc = acc_ref[0] + acc_ref[1] + hs_ref[...]
    conv = dinv * sacc + b_ref[...]
    mean = jnp.mean(conv, axis=0, keepdims=True)
    xc = conv - ga_ref[...] * mean
    var = jnp.mean(xc * xc, axis=0, keepdims=True)
    g = gw_ref[...] * xc * lax.rsqrt(var + 1e-5) + gb_ref[...]
    hsn_ref[...] = dinv * jnp.dot(g, wn_ref[...],
                                  preferred_element_type=jnp.float32)


def _tc_last_body(acc_ref, hs_ref, dinv_ref, b_ref, gw_ref, gb_ref, ga_ref,
                  out_ref):
    sacc = acc_ref[0] + acc_ref[1] + hs_ref[...]
    conv = dinv_ref[...] * sacc + b_ref[...]
    mean = jnp.mean(conv, axis=0, keepdims=True)
    xc = conv - ga_ref[...] * mean
    var = jnp.mean(xc * xc, axis=0, keepdims=True)
    out_ref[...] = gw_ref[...] * xc * lax.rsqrt(var + 1e-5) + gb_ref[...]


_f32 = jnp.float32
_tc_first = pl.pallas_call(
    _tc_first_body,
    out_shape=[jax.ShapeDtypeStruct((N, 1), _f32),
               jax.ShapeDtypeStruct((N, D_H), _f32)],
)
_tc_mid = pl.pallas_call(
    _tc_mid_body,
    out_shape=jax.ShapeDtypeStruct((N, D_H), _f32),
)
_tc_last = pl.pallas_call(
    _tc_last_body,
    out_shape=jax.ShapeDtypeStruct((N, D_H), _f32),
)


def kernel(x, edge_index, W0, b0, gw0, gb0, ga0, W1, b1, gw1, gb1, ga1,
           W2, b2, gw2, gb2, ga2):
    pad = E_PAD - E
    pidx = jnp.arange(pad, dtype=jnp.int32)
    src_r = jnp.concatenate(
        [edge_index[0], pidx % N]).reshape(NW, NJ, K)
    dst_r = jnp.concatenate(
        [edge_index[1], N + pidx % (NP - N)]).reshape(NW, NJ, K)

    histp = _deg_kernel(dst_r)
    dinv, hs = _tc_first(histp, x, W0)

    params = [(b0, gw0, gb0, ga0), (b1, gw1, gb1, ga1), (b2, gw2, gb2, ga2)]
    row = lambda v: v.reshape(1, D_H)

    for layer in range(3):
        acc = _edge_kernel(hs, src_r, dst_r)
        b, gw, gb, ga = (row(v) for v in params[layer])
        if layer < 2:
            wn = (W1, W2)[layer]
            hs = _tc_mid(acc, hs, dinv, b, gw, gb, ga, wn)
        else:
            out = _tc_last(acc, hs, dinv, b, gw, gb, ga)
    return out


# speedup vs baseline: 2.2481x; 1.0917x over previous
import functools

import jax
import jax.numpy as jnp
from jax import lax
from jax.experimental import pallas as pl
from jax.experimental.pallas import tpu as pltpu
from jax.experimental.pallas import tpu_sc as plsc

N = 10000
E = 320000
D_IN = 128
D_H = 64

NC = 2
NS = 16
NW = NC * NS
K = 80
NJ = 126
EPW = NJ * K
E_PAD = NW * EPW
NP = 10240
RPT = 640
L = 16
ZC = 80
NBUF = 6
DG = 3
DS = 3

_mesh = plsc.VectorSubcoreMesh(core_axis_name="c", subcore_axis_name="s")
_sc_params = pltpu.CompilerParams(use_tc_tiling_on_sc=False)


@functools.partial(
    pl.kernel,
    out_type=jax.ShapeDtypeStruct((NC, N, L), jnp.float32),
    mesh=_mesh,
    compiler_params=_sc_params,
    scratch_types=[
        pltpu.VMEM((NJ, K), jnp.int32),
        pltpu.VMEM((K, L), jnp.float32),
        pltpu.VMEM_SHARED((NP, L), jnp.float32),
    ],
)
def _deg_kernel(dst_hbm, out_hbm, dst_v, ones_v, acc):
    c = lax.axis_index("c")
    s = lax.axis_index("s")
    w = s * NC + c
    pltpu.sync_copy(dst_hbm.at[w], dst_v)

    def fill(i, carry):
        ones_v[i, :] = jnp.full((L,), carry, jnp.float32)
        return carry

    lax.fori_loop(0, K, fill, 0.0)
    base = s * RPT
    for m in range(RPT // ZC):
        if (m + 1) * ZC <= 400:
            pltpu.sync_copy(ones_v.at[pl.ds(0, ZC)],
                            acc.at[pl.ds(base + m * ZC, ZC)])
        else:
            @pl.when(s < NS - 1)
            def _():
                pltpu.sync_copy(ones_v.at[pl.ds(0, ZC)],
                                acc.at[pl.ds(base + m * ZC, ZC)])
    lax.fori_loop(0, K, fill, 1.0)
    plsc.subcore_barrier()

    def body(j, carry):
        pltpu.sync_copy(ones_v, acc.at[dst_v.at[j]], add=True)
        return carry

    lax.fori_loop(0, NJ, body, 0)
    plsc.subcore_barrier()

    @pl.when(s < NS - 1)
    def _():
        pltpu.sync_copy(acc.at[pl.ds(base, RPT)], out_hbm.at[c, pl.ds(base, RPT)])

    @pl.when(s == NS - 1)
    def _():
        pltpu.sync_copy(acc.at[pl.ds(N - 400, 400)],
                        out_hbm.at[c, pl.ds(N - 400, 400)])


@functools.partial(
    pl.kernel,
    out_type=jax.ShapeDtypeStruct((NC, N, D_H), jnp.float32),
    mesh=_mesh,
    compiler_params=_sc_params,
    scratch_types=[
        pltpu.VMEM((NJ, K), jnp.int32),
        pltpu.VMEM((NJ, K), jnp.int32),
        [pltpu.VMEM((K, D_H), jnp.float32)] * NBUF,
        pltpu.VMEM_SHARED((NP, D_H), jnp.float32),
        [pltpu.SemaphoreType.DMA] * NBUF,
        [pltpu.SemaphoreType.DMA] * NBUF,
    ],
)
def _edge_kernel(hs_hbm, src_hbm, dst_hbm, out_hbm, src_v, dst_v, rows,
                 acc, semg, sems):
    c = lax.axis_index("c")
    s = lax.axis_index("s")
    w = s * NC + c

    pltpu.sync_copy(src_hbm.at[w], src_v)
    pltpu.sync_copy(dst_hbm.at[w], dst_v)

    zero = jnp.zeros((L,), jnp.float32)

    def zbody(i, carry):
        def zcol(k2, carry2):
            rows[0][i, pl.ds(k2 * L, L)] = zero
            return carry2

        return lax.fori_loop(0, D_H // L, zcol, carry)

    lax.fori_loop(0, K, zbody, 0)

    base = s * RPT
    for m in range(RPT // ZC):
        if (m + 1) * ZC <= 400:
            pltpu.sync_copy(rows[0].at[pl.ds(0, ZC)],
                            acc.at[pl.ds(base + m * ZC, ZC)])
        else:
            @pl.when(s < NS - 1)
            def _():
                pltpu.sync_copy(rows[0].at[pl.ds(0, ZC)],
                                acc.at[pl.ds(base + m * ZC, ZC)])
    plsc.subcore_barrier()

    def gwait(j, b):
        pltpu.make_async_copy(hs_hbm.at[src_v.at[j]], rows[b], semg[b]).wait()

    def swait(j, b):
        pltpu.make_async_copy(rows[b], acc.at[dst_v.at[j]], sems[b]).wait()

    def sfire(j, b):
        pltpu.async_copy(rows[b], acc.at[dst_v.at[j]], sems[b], add=True)

    def gfire(j, b):
        pltpu.async_copy(hs_hbm.at[src_v.at[j]], rows[b], semg[b])

    assert (NJ - DS - DG) % NBUF == 0
    for b in range(DG):
        gfire(b, b)
    for p in range(DS):
        gwait(p, p % NBUF)
        sfire(p, p % NBUF)
        gfire(p + DG, (p + DG) % NBUF)

    def body(i, carry):
        p0 = DS + NBUF * i
        for u in range(NBUF):
            b = (DS + u) % NBUF
            bn = (b + DG) % NBUF
            gwait(p0 + u, b)
            sfire(p0 + u, b)
            swait(p0 + u - DS, bn)
            gfire(p0 + u + DG, bn)
        return carry

    lax.fori_loop(0, (NJ - DS - DG) // NBUF, body, 0)
    for p in range(NJ - DG, NJ):
        b = p % NBUF
        gwait(p, b)
        sfire(p, b)
        swait(p - DS, (b + DG) % NBUF)
    for p in range(NJ - DS, NJ):
        swait(p, p % NBUF)
    plsc.subcore_barrier()

    @pl.when(s < NS - 1)
    def _():
        pltpu.sync_copy(acc.at[pl.ds(base, RPT)], out_hbm.at[c, pl.ds(base, RPT)])

    @pl.when(s == NS - 1)
    def _():
        pltpu.sync_copy(acc.at[pl.ds(N - 400, 400)],
                        out_hbm.at[c, pl.ds(N - 400, 400)])


def _tc_first_body(hist_ref, x_ref, w0_ref, dinv_ref, hs_ref):
    deg = hist_ref[0, :, 0:1] + hist_ref[1, :, 0:1] + 1.0
    dinv = lax.rsqrt(deg)
    h = jnp.dot(x_ref[...], w0_ref[...], preferred_element_type=jnp.float32)
    dinv_ref[...] = dinv
    hs_ref[...] = dinv * h


def _tc_mid_body(acc_ref, hs_ref, dinv_ref, b_ref, gw_ref, gb_ref, ga_ref,
                 wn_ref, hsn_ref):
    dinv = dinv_ref[...]
    sacc = acc_ref[0] + acc_ref[1] + hs_ref[...]
    conv = dinv * sacc + b_ref[...]
    mean = jnp.mean(conv, axis=0, keepdims=True)
    xc = conv - ga_ref[...] * mean
    var = jnp.mean(xc * xc, axis=0, keepdims=True)
    g = gw_ref[...] * xc * lax.rsqrt(var + 1e-5) + gb_ref[...]
    hsn_ref[...] = dinv * jnp.dot(g, wn_ref[...],
                                  preferred_element_type=jnp.float32)


def _tc_last_body(acc_ref, hs_ref, dinv_ref, b_ref, gw_ref, gb_ref, ga_ref,
                  out_ref):
    sacc = acc_ref[0] + acc_ref[1] + hs_ref[...]
    conv = dinv_ref[...] * sacc + b_ref[...]
    mean = jnp.mean(conv, axis=0, keepdims=True)
    xc = conv - ga_ref[...] * mean
    var = jnp.mean(xc * xc, axis=0, keepdims=True)
    out_ref[...] = gw_ref[...] * xc * lax.rsqrt(var + 1e-5) + gb_ref[...]


_f32 = jnp.float32
_tc_first = pl.pallas_call(
    _tc_first_body,
    out_shape=[jax.ShapeDtypeStruct((N, 1), _f32),
               jax.ShapeDtypeStruct((N, D_H), _f32)],
)
_tc_mid = pl.pallas_call(
    _tc_mid_body,
    out_shape=jax.ShapeDtypeStruct((N, D_H), _f32),
)
_tc_last = pl.pallas_call(
    _tc_last_body,
    out_shape=jax.ShapeDtypeStruct((N, D_H), _f32),
)


def kernel(x, edge_index, W0, b0, gw0, gb0, ga0, W1, b1, gw1, gb1, ga1,
           W2, b2, gw2, gb2, ga2):
    pad = E_PAD - E
    pidx = jnp.arange(pad, dtype=jnp.int32)
    src_r = jnp.concatenate(
        [edge_index[0], pidx % N]).reshape(NW, NJ, K)
    dst_r = jnp.concatenate(
        [edge_index[1], N + pidx % (NP - N)]).reshape(NW, NJ, K)

    histp = _deg_kernel(dst_r)
    dinv, hs = _tc_first(histp, x, W0)

    params = [(b0, gw0, gb0, ga0), (b1, gw1, gb1, ga1), (b2, gw2, gb2, ga2)]
    row = lambda v: v.reshape(1, D_H)

    for layer in range(3):
        acc = _edge_kernel(hs, src_r, dst_r)
        b, gw, gb, ga = (row(v) for v in params[layer])
        if layer < 2:
            wn = (W1, W2)[layer]
            hs = _tc_mid(acc, hs, dinv, b, gw, gb, ga, wn)
        else:
            out = _tc_last(acc, hs, dinv, b, gw, gb, ga)
    return out


# speedup vs baseline: 2.3087x; 1.0269x over previous
import functools

import jax
import jax.numpy as jnp
from jax import lax
from jax.experimental import pallas as pl
from jax.experimental.pallas import tpu as pltpu
from jax.experimental.pallas import tpu_sc as plsc

N = 10000
E = 320000
D_IN = 128
D_H = 64

NC = 2
NS = 16
NW = NC * NS
K = 80
NJ = 128
EPW = NJ * K
E_PAD = NW * EPW
NP = 10240
RPT = 640
L = 16
ZC = 80
NBUF = 8
DG = 4
DS = 4

_mesh = plsc.VectorSubcoreMesh(core_axis_name="c", subcore_axis_name="s")
_sc_params = pltpu.CompilerParams(use_tc_tiling_on_sc=False)


@functools.partial(
    pl.kernel,
    out_type=jax.ShapeDtypeStruct((NC, N, L), jnp.float32),
    mesh=_mesh,
    compiler_params=_sc_params,
    scratch_types=[
        pltpu.VMEM((NJ, K), jnp.int32),
        pltpu.VMEM((K, L), jnp.float32),
        pltpu.VMEM_SHARED((NP, L), jnp.float32),
    ],
)
def _deg_kernel(dst_hbm, out_hbm, dst_v, ones_v, acc):
    c = lax.axis_index("c")
    s = lax.axis_index("s")
    w = s * NC + c
    pltpu.sync_copy(dst_hbm.at[w], dst_v)

    def fill(i, carry):
        ones_v[i, :] = jnp.full((L,), carry, jnp.float32)
        return carry

    lax.fori_loop(0, K, fill, 0.0)
    base = s * RPT
    for m in range(RPT // ZC):
        if (m + 1) * ZC <= 400:
            pltpu.sync_copy(ones_v.at[pl.ds(0, ZC)],
                            acc.at[pl.ds(base + m * ZC, ZC)])
        else:
            @pl.when(s < NS - 1)
            def _():
                pltpu.sync_copy(ones_v.at[pl.ds(0, ZC)],
                                acc.at[pl.ds(base + m * ZC, ZC)])
    lax.fori_loop(0, K, fill, 1.0)
    plsc.subcore_barrier()

    def body(j, carry):
        pltpu.sync_copy(ones_v, acc.at[dst_v.at[j]], add=True)
        return carry

    lax.fori_loop(0, NJ, body, 0)
    plsc.subcore_barrier()

    @pl.when(s < NS - 1)
    def _():
        pltpu.sync_copy(acc.at[pl.ds(base, RPT)], out_hbm.at[c, pl.ds(base, RPT)])

    @pl.when(s == NS - 1)
    def _():
        pltpu.sync_copy(acc.at[pl.ds(N - 400, 400)],
                        out_hbm.at[c, pl.ds(N - 400, 400)])


@functools.partial(
    pl.kernel,
    out_type=jax.ShapeDtypeStruct((NC, N, D_H), jnp.float32),
    mesh=_mesh,
    compiler_params=_sc_params,
    scratch_types=[
        pltpu.VMEM((NJ, K), jnp.int32),
        pltpu.VMEM((NJ, K), jnp.int32),
        [pltpu.VMEM((K, D_H), jnp.float32)] * NBUF,
        pltpu.VMEM_SHARED((NP, D_H), jnp.float32),
        [pltpu.SemaphoreType.DMA] * NBUF,
        [pltpu.SemaphoreType.DMA] * NBUF,
    ],
)
def _edge_kernel(hs_hbm, src_hbm, dst_hbm, out_hbm, src_v, dst_v, rows,
                 acc, semg, sems):
    c = lax.axis_index("c")
    s = lax.axis_index("s")
    w = s * NC + c

    pltpu.sync_copy(src_hbm.at[w], src_v)
    pltpu.sync_copy(dst_hbm.at[w], dst_v)

    zero = jnp.zeros((L,), jnp.float32)

    def zbody(i, carry):
        def zcol(k2, carry2):
            rows[0][i, pl.ds(k2 * L, L)] = zero
            return carry2

        return lax.fori_loop(0, D_H // L, zcol, carry)

    lax.fori_loop(0, K, zbody, 0)

    base = s * RPT
    for m in range(RPT // ZC):
        if (m + 1) * ZC <= 400:
            pltpu.sync_copy(rows[0].at[pl.ds(0, ZC)],
                            acc.at[pl.ds(base + m * ZC, ZC)])
        else:
            @pl.when(s < NS - 1)
            def _():
                pltpu.sync_copy(rows[0].at[pl.ds(0, ZC)],
                                acc.at[pl.ds(base + m * ZC, ZC)])
    plsc.subcore_barrier()

    def gwait(j, b):
        pltpu.make_async_copy(hs_hbm.at[src_v.at[j]], rows[b], semg[b]).wait()

    def swait(j, b):
        pltpu.make_async_copy(rows[b], acc.at[dst_v.at[j]], sems[b]).wait()

    def sfire(j, b):
        pltpu.async_copy(rows[b], acc.at[dst_v.at[j]], sems[b], add=True)

    def gfire(j, b):
        pltpu.async_copy(hs_hbm.at[src_v.at[j]], rows[b], semg[b])

    assert (NJ - DS - DG) % NBUF == 0
    for b in range(DG):
        gfire(b, b)
    for p in range(DS):
        gwait(p, p % NBUF)
        sfire(p, p % NBUF)
        gfire(p + DG, (p + DG) % NBUF)

    def body(i, carry):
        p0 = DS + NBUF * i
        for u in range(NBUF):
            b = (DS + u) % NBUF
            bn = (b + DG) % NBUF
            gwait(p0 + u, b)
            sfire(p0 + u, b)
            swait(p0 + u - DS, bn)
            gfire(p0 + u + DG, bn)
        return carry

    lax.fori_loop(0, (NJ - DS - DG) // NBUF, body, 0)
    for p in range(NJ - DG, NJ):
        b = p % NBUF
        gwait(p, b)
        sfire(p, b)
        swait(p - DS, (b + DG) % NBUF)
    for p in range(NJ - DS, NJ):
        swait(p, p % NBUF)
    plsc.subcore_barrier()

    @pl.when(s < NS - 1)
    def _():
        pltpu.sync_copy(acc.at[pl.ds(base, RPT)], out_hbm.at[c, pl.ds(base, RPT)])

    @pl.when(s == NS - 1)
    def _():
        pltpu.sync_copy(acc.at[pl.ds(N - 400, 400)],
                        out_hbm.at[c, pl.ds(N - 400, 400)])


def _tc_first_body(hist_ref, x_ref, w0_ref, dinv_ref, hs_ref):
    deg = hist_ref[0, :, 0:1] + hist_ref[1, :, 0:1] + 1.0
    dinv = lax.rsqrt(deg)
    h = jnp.dot(x_ref[...], w0_ref[...], preferred_element_type=jnp.float32)
    dinv_ref[...] = dinv
    hs_ref[...] = dinv * h


def _tc_mid_body(acc_ref, hs_ref, dinv_ref, b_ref, gw_ref, gb_ref, ga_ref,
                 wn_ref, hsn_ref):
    dinv = dinv_ref[...]
    sacc = acc_ref[0] + acc_ref[1] + hs_ref[...]
    conv = dinv * sacc + b_ref[...]
    mean = jnp.mean(conv, axis=0, keepdims=True)
    xc = conv - ga_ref[...] * mean
    var = jnp.mean(xc * xc, axis=0, keepdims=True)
    g = gw_ref[...] * xc * lax.rsqrt(var + 1e-5) + gb_ref[...]
    hsn_ref[...] = dinv * jnp.dot(g, wn_ref[...],
                                  preferred_element_type=jnp.float32)


def _tc_last_body(acc_ref, hs_ref, dinv_ref, b_ref, gw_ref, gb_ref, ga_ref,
                  out_ref):
    sacc = acc_ref[0] + acc_ref[1] + hs_ref[...]
    conv = dinv_ref[...] * sacc + b_ref[...]
    mean = jnp.mean(conv, axis=0, keepdims=True)
    xc = conv - ga_ref[...] * mean
    var = jnp.mean(xc * xc, axis=0, keepdims=True)
    out_ref[...] = gw_ref[...] * xc * lax.rsqrt(var + 1e-5) + gb_ref[...]


_f32 = jnp.float32
_tc_first = pl.pallas_call(
    _tc_first_body,
    out_shape=[jax.ShapeDtypeStruct((N, 1), _f32),
               jax.ShapeDtypeStruct((N, D_H), _f32)],
)
_tc_mid = pl.pallas_call(
    _tc_mid_body,
    out_shape=jax.ShapeDtypeStruct((N, D_H), _f32),
)
_tc_last = pl.pallas_call(
    _tc_last_body,
    out_shape=jax.ShapeDtypeStruct((N, D_H), _f32),
)


def kernel(x, edge_index, W0, b0, gw0, gb0, ga0, W1, b1, gw1, gb1, ga1,
           W2, b2, gw2, gb2, ga2):
    pad = E_PAD - E
    pidx = jnp.arange(pad, dtype=jnp.int32)
    src_r = jnp.concatenate(
        [edge_index[0], pidx % N]).reshape(NW, NJ, K)
    dst_r = jnp.concatenate(
        [edge_index[1], N + pidx % (NP - N)]).reshape(NW, NJ, K)

    histp = _deg_kernel(dst_r)
    dinv, hs = _tc_first(histp, x, W0)

    params = [(b0, gw0, gb0, ga0), (b1, gw1, gb1, ga1), (b2, gw2, gb2, ga2)]
    row = lambda v: v.reshape(1, D_H)

    for layer in range(3):
        acc = _edge_kernel(hs, src_r, dst_r)
        b, gw, gb, ga = (row(v) for v in params[layer])
        if layer < 2:
            wn = (W1, W2)[layer]
            hs = _tc_mid(acc, hs, dinv, b, gw, gb, ga, wn)
        else:
            out = _tc_last(acc, hs, dinv, b, gw, gb, ga)
    return out
